# baseline (speedup 1.0000x reference)
import jax
import jax.numpy as jnp
from jax.experimental import pallas as pl
from jax.experimental.pallas import tpu as pltpu

_K = 8


def _dma_copy(x_hbm, o_hbm, sems):
    n = x_hbm.shape[0]
    c = (n // _K) // 8 * 8
    rem = n - _K * c
    copies = [
        pltpu.make_async_copy(
            x_hbm.at[pl.ds(k * c, c), :],
            o_hbm.at[pl.ds(k * c, c), :],
            sems.at[k],
        )
        for k in range(_K)
    ]
    if rem:
        copies.append(
            pltpu.make_async_copy(
                x_hbm.at[pl.ds(_K * c, rem), :],
                o_hbm.at[pl.ds(_K * c, rem), :],
                sems.at[_K],
            )
        )
    for cp in copies:
        cp.start()
    for cp in copies:
        cp.wait()


def kernel(x, u):
    n, d = x.shape
    return pl.pallas_call(
        _dma_copy,
        in_specs=[pl.BlockSpec(memory_space=pl.ANY)],
        out_specs=pl.BlockSpec(memory_space=pl.ANY),
        out_shape=jax.ShapeDtypeStruct((n, d), x.dtype),
        scratch_shapes=[pltpu.SemaphoreType.DMA((_K + 1,))],
    )(x)
